# Initial kernel scaffold; baseline (speedup 1.0000x reference)
import jax
import jax.numpy as jnp
from jax.experimental import pallas as pl


def kernel(xyz1, xyz2):
    raise NotImplementedError("write your pallas kernel here")


# speedup vs baseline: 1.1739x; 1.1739x over previous
import functools

import jax
import jax.numpy as jnp
from jax.experimental import pallas as pl
from jax.experimental.pallas import tpu as pltpu

_EPS = 1e-12
_BIG_I32 = 2**30


def _chamfer_body(p1_ref, n1_ref, p2t_ref, n2t_ref,
                  d1_ref, nd1_ref, d2_ref, nd2_ref,
                  accd_ref, accn_ref):
    mt = pl.program_id(1)
    nmt = pl.num_programs(1)

    p1 = p1_ref[0]
    n1 = n1_ref[0]
    p2t = p2t_ref[0]
    n2t = n2t_ref[0]

    sq1 = jnp.sum(p1 * p1, axis=1, keepdims=True)
    sq2 = jnp.sum(p2t * p2t, axis=0, keepdims=True)
    inner = jnp.dot(p1, p2t, preferred_element_type=jnp.float32)
    d = sq1 + sq2 - 2.0 * inner

    rmin = jnp.min(d, axis=1, keepdims=True)
    iota_m = jax.lax.broadcasted_iota(jnp.int32, d.shape, 1)
    rarg = jnp.min(jnp.where(d == rmin, iota_m, _BIG_I32), axis=1, keepdims=True)
    oh1 = (iota_m == rarg).astype(jnp.float32)
    cand_n = jnp.concatenate(
        [jnp.sum(oh1 * n2t[k:k + 1, :], axis=1, keepdims=True) for k in range(3)],
        axis=1)

    @pl.when(mt == 0)
    def _():
        accd_ref[...] = rmin
        accn_ref[...] = cand_n

    @pl.when(mt > 0)
    def _():
        prev = accd_ref[...]
        upd = rmin < prev
        accd_ref[...] = jnp.where(upd, rmin, prev)
        accn_ref[...] = jnp.where(upd, cand_n, accn_ref[...])

    cmin = jnp.min(d, axis=0, keepdims=True)
    iota_n = jax.lax.broadcasted_iota(jnp.int32, d.shape, 0)
    carg = jnp.min(jnp.where(d == cmin, iota_n, _BIG_I32), axis=0, keepdims=True)
    oh2 = (iota_n == carg).astype(jnp.float32)
    tn2 = [jnp.sum(oh2 * n1[:, k:k + 1], axis=0, keepdims=True) for k in range(3)]

    d2_ref[0] = cmin

    nsq2 = n2t[0:1] ** 2 + n2t[1:2] ** 2 + n2t[2:3] ** 2
    inv2 = 1.0 / jnp.maximum(jnp.sqrt(nsq2), _EPS)
    tsq2 = tn2[0] ** 2 + tn2[1] ** 2 + tn2[2] ** 2
    invt2 = 1.0 / jnp.maximum(jnp.sqrt(tsq2), _EPS)
    nd2_ref[0] = ((n2t[0:1] * inv2 - tn2[0] * invt2) ** 2
                  + (n2t[1:2] * inv2 - tn2[1] * invt2) ** 2
                  + (n2t[2:3] * inv2 - tn2[2] * invt2) ** 2)

    @pl.when(mt == nmt - 1)
    def _():
        an = accn_ref[...]
        d1_ref[0] = accd_ref[...]
        inv1 = 1.0 / jnp.maximum(
            jnp.sqrt(jnp.sum(n1 * n1, axis=1, keepdims=True)), _EPS)
        invt1 = 1.0 / jnp.maximum(
            jnp.sqrt(jnp.sum(an * an, axis=1, keepdims=True)), _EPS)
        diff = n1 * inv1 - an * invt1
        nd1_ref[0] = jnp.sum(diff * diff, axis=1, keepdims=True)


@functools.partial(jax.jit, static_argnames=("tm",))
def _chamfer(xyz1, xyz2, tm=512):
    B, N, _ = xyz1.shape
    M = xyz2.shape[1]
    p1 = xyz1[:, :, :3]
    n1 = xyz1[:, :, 3:]
    p2t = jnp.transpose(xyz2[:, :, :3], (0, 2, 1))
    n2t = jnp.transpose(xyz2[:, :, 3:], (0, 2, 1))

    grid = (B, M // tm)
    d1, nd1, d2, nd2 = pl.pallas_call(
        _chamfer_body,
        grid=grid,
        in_specs=[
            pl.BlockSpec((1, N, 3), lambda b, m: (b, 0, 0)),
            pl.BlockSpec((1, N, 3), lambda b, m: (b, 0, 0)),
            pl.BlockSpec((1, 3, tm), lambda b, m: (b, 0, m)),
            pl.BlockSpec((1, 3, tm), lambda b, m: (b, 0, m)),
        ],
        out_specs=[
            pl.BlockSpec((1, N, 1), lambda b, m: (b, 0, 0)),
            pl.BlockSpec((1, N, 1), lambda b, m: (b, 0, 0)),
            pl.BlockSpec((1, 1, tm), lambda b, m: (b, 0, m)),
            pl.BlockSpec((1, 1, tm), lambda b, m: (b, 0, m)),
        ],
        out_shape=[
            jax.ShapeDtypeStruct((B, N, 1), jnp.float32),
            jax.ShapeDtypeStruct((B, N, 1), jnp.float32),
            jax.ShapeDtypeStruct((B, 1, M), jnp.float32),
            jax.ShapeDtypeStruct((B, 1, M), jnp.float32),
        ],
        scratch_shapes=[
            pltpu.VMEM((N, 1), jnp.float32),
            pltpu.VMEM((N, 3), jnp.float32),
        ],
    )(p1, n1, p2t, n2t)
    return jnp.mean(d1) + jnp.mean(d2) + jnp.mean(nd1) + jnp.mean(nd2)


def kernel(xyz1, xyz2):
    return _chamfer(xyz1, xyz2)


# speedup vs baseline: 1.3897x; 1.1838x over previous
import functools

import jax
import jax.numpy as jnp
from jax.experimental import pallas as pl
from jax.experimental.pallas import tpu as pltpu

_EPS = 1e-12


def _chamfer_body(a1_ref, a1t_ref, a2r_ref, a2t_ref, n1_ref, n2r_ref,
                  d1_ref, nd1_ref, d2_ref, nd2_ref,
                  accd_ref, accn_ref):
    mt = pl.program_id(1)
    nmt = pl.num_programs(1)

    a1 = a1_ref[0]
    a1t = a1t_ref[0]
    a2r = a2r_ref[0]
    a2t = a2t_ref[0]
    n1 = n1_ref[0]
    n2r = n2r_ref[0]

    d = jnp.dot(a1, a2t, preferred_element_type=jnp.float32)
    dT = jnp.dot(a2r, a1t, preferred_element_type=jnp.float32)

    rmin = jnp.min(d, axis=1, keepdims=True)
    oh1 = (d == rmin).astype(jnp.float32)
    cand_n = jnp.dot(oh1, n2r, preferred_element_type=jnp.float32)

    @pl.when(mt == 0)
    def _():
        accd_ref[...] = rmin
        accn_ref[...] = cand_n

    @pl.when(mt > 0)
    def _():
        prev = accd_ref[...]
        upd = rmin < prev
        accd_ref[...] = jnp.where(upd, rmin, prev)
        accn_ref[...] = jnp.where(upd, cand_n, accn_ref[...])

    cmin = jnp.min(dT, axis=1, keepdims=True)
    oh2 = (dT == cmin).astype(jnp.float32)
    tn2 = jnp.dot(oh2, n1, preferred_element_type=jnp.float32)

    d2_ref[0] = cmin

    inv2 = 1.0 / jnp.maximum(
        jnp.sqrt(jnp.sum(n2r * n2r, axis=1, keepdims=True)), _EPS)
    invt2 = 1.0 / jnp.maximum(
        jnp.sqrt(jnp.sum(tn2 * tn2, axis=1, keepdims=True)), _EPS)
    diff2 = n2r * inv2 - tn2 * invt2
    nd2_ref[0] = jnp.sum(diff2 * diff2, axis=1, keepdims=True)

    @pl.when(mt == nmt - 1)
    def _():
        an = accn_ref[...]
        d1_ref[0] = accd_ref[...]
        inv1 = 1.0 / jnp.maximum(
            jnp.sqrt(jnp.sum(n1 * n1, axis=1, keepdims=True)), _EPS)
        invt1 = 1.0 / jnp.maximum(
            jnp.sqrt(jnp.sum(an * an, axis=1, keepdims=True)), _EPS)
        diff = n1 * inv1 - an * invt1
        nd1_ref[0] = jnp.sum(diff * diff, axis=1, keepdims=True)


@functools.partial(jax.jit, static_argnames=("tm",))
def _chamfer(xyz1, xyz2, tm=512):
    B, N, _ = xyz1.shape
    M = xyz2.shape[1]
    f32 = jnp.float32

    p1 = xyz1[:, :, :3]
    n1 = xyz1[:, :, 3:]
    p2 = xyz2[:, :, :3]
    n2 = xyz2[:, :, 3:]
    sq1 = jnp.sum(p1 * p1, axis=2, keepdims=True)
    sq2 = jnp.sum(p2 * p2, axis=2, keepdims=True)
    z1 = jnp.zeros((B, N, 3), f32)
    z2 = jnp.zeros((B, M, 3), f32)
    a1 = jnp.concatenate([p1, sq1, jnp.ones((B, N, 1), f32), z1], axis=2)
    a2 = jnp.concatenate([-2.0 * p2, jnp.ones((B, M, 1), f32), sq2, z2], axis=2)
    a1t = jnp.transpose(a1, (0, 2, 1))
    a2t = jnp.transpose(a2, (0, 2, 1))

    grid = (B, M // tm)
    d1, nd1, d2, nd2 = pl.pallas_call(
        _chamfer_body,
        grid=grid,
        in_specs=[
            pl.BlockSpec((1, N, 8), lambda b, m: (b, 0, 0)),
            pl.BlockSpec((1, 8, N), lambda b, m: (b, 0, 0)),
            pl.BlockSpec((1, tm, 8), lambda b, m: (b, m, 0)),
            pl.BlockSpec((1, 8, tm), lambda b, m: (b, 0, m)),
            pl.BlockSpec((1, N, 3), lambda b, m: (b, 0, 0)),
            pl.BlockSpec((1, tm, 3), lambda b, m: (b, m, 0)),
        ],
        out_specs=[
            pl.BlockSpec((1, N, 1), lambda b, m: (b, 0, 0)),
            pl.BlockSpec((1, N, 1), lambda b, m: (b, 0, 0)),
            pl.BlockSpec((1, tm, 1), lambda b, m: (b, m, 0)),
            pl.BlockSpec((1, tm, 1), lambda b, m: (b, m, 0)),
        ],
        out_shape=[
            jax.ShapeDtypeStruct((B, N, 1), f32),
            jax.ShapeDtypeStruct((B, N, 1), f32),
            jax.ShapeDtypeStruct((B, M, 1), f32),
            jax.ShapeDtypeStruct((B, M, 1), f32),
        ],
        scratch_shapes=[
            pltpu.VMEM((N, 1), f32),
            pltpu.VMEM((N, 3), f32),
        ],
    )(a1, a1t, a2, a2t, n1, n2)
    return jnp.mean(d1) + jnp.mean(d2) + jnp.mean(nd1) + jnp.mean(nd2)


def kernel(xyz1, xyz2):
    return _chamfer(xyz1, xyz2)


# speedup vs baseline: 1.9534x; 1.4057x over previous
import functools

import jax
import jax.numpy as jnp
from jax.experimental import pallas as pl
from jax.experimental.pallas import tpu as pltpu

_EPS = 1e-12


def _chamfer_body(a2r_ref, a1t_ref, n1_ref, n1t_ref, n2t_ref, n2r_ref,
                  d1_ref, nd1_ref, d2_ref, nd2_ref,
                  accd_ref, accn_ref):
    mt = pl.program_id(1)
    nmt = pl.num_programs(1)

    a2r = a2r_ref[0]
    a1t = a1t_ref[0]
    n1 = n1_ref[0]
    n1t = n1t_ref[0]
    n2t = n2t_ref[0]
    n2r = n2r_ref[0]

    dT = jnp.dot(a2r, a1t, preferred_element_type=jnp.float32)

    rmin = jnp.min(dT, axis=0, keepdims=True)
    oh1 = (dT == rmin).astype(jnp.float32)
    cand_n = jnp.dot(n2t, oh1, preferred_element_type=jnp.float32)

    @pl.when(mt == 0)
    def _():
        accd_ref[...] = rmin
        accn_ref[...] = cand_n

    @pl.when(mt > 0)
    def _():
        prev = accd_ref[...]
        upd = rmin < prev
        accd_ref[...] = jnp.where(upd, rmin, prev)
        accn_ref[...] = jnp.where(upd, cand_n, accn_ref[...])

    cmin = jnp.min(dT, axis=1, keepdims=True)
    oh2 = (dT == cmin).astype(jnp.float32)
    tn2 = jnp.dot(oh2, n1, preferred_element_type=jnp.float32)

    d2_ref[0] = cmin

    inv2 = 1.0 / jnp.maximum(
        jnp.sqrt(jnp.sum(n2r * n2r, axis=1, keepdims=True)), _EPS)
    invt2 = 1.0 / jnp.maximum(
        jnp.sqrt(jnp.sum(tn2 * tn2, axis=1, keepdims=True)), _EPS)
    diff2 = n2r * inv2 - tn2 * invt2
    nd2_ref[0] = jnp.sum(diff2 * diff2, axis=1, keepdims=True)

    @pl.when(mt == nmt - 1)
    def _():
        an = accn_ref[...]
        d1_ref[0] = accd_ref[...]
        inv1 = 1.0 / jnp.maximum(
            jnp.sqrt(jnp.sum(n1t * n1t, axis=0, keepdims=True)), _EPS)
        invt1 = 1.0 / jnp.maximum(
            jnp.sqrt(jnp.sum(an * an, axis=0, keepdims=True)), _EPS)
        diff = n1t * inv1 - an * invt1
        nd1_ref[0] = jnp.sum(diff * diff, axis=0, keepdims=True)


@functools.partial(jax.jit, static_argnames=("tm",))
def _chamfer(xyz1, xyz2, tm=512):
    B, N, _ = xyz1.shape
    M = xyz2.shape[1]
    f32 = jnp.float32

    p1 = xyz1[:, :, :3]
    n1 = xyz1[:, :, 3:]
    p2 = xyz2[:, :, :3]
    n2 = xyz2[:, :, 3:]
    sq1 = jnp.sum(p1 * p1, axis=2, keepdims=True)
    sq2 = jnp.sum(p2 * p2, axis=2, keepdims=True)
    a1 = jnp.concatenate([p1, sq1, jnp.ones((B, N, 1), f32),
                          jnp.zeros((B, N, 3), f32)], axis=2)
    a2 = jnp.concatenate([-2.0 * p2, jnp.ones((B, M, 1), f32), sq2,
                          jnp.zeros((B, M, 3), f32)], axis=2)
    a1t = jnp.transpose(a1, (0, 2, 1))
    n1t = jnp.transpose(n1, (0, 2, 1))
    n2t = jnp.transpose(n2, (0, 2, 1))

    grid = (B, M // tm)
    d1, nd1, d2, nd2 = pl.pallas_call(
        _chamfer_body,
        grid=grid,
        in_specs=[
            pl.BlockSpec((1, tm, 8), lambda b, m: (b, m, 0)),
            pl.BlockSpec((1, 8, N), lambda b, m: (b, 0, 0)),
            pl.BlockSpec((1, N, 3), lambda b, m: (b, 0, 0)),
            pl.BlockSpec((1, 3, N), lambda b, m: (b, 0, 0)),
            pl.BlockSpec((1, 3, tm), lambda b, m: (b, 0, m)),
            pl.BlockSpec((1, tm, 3), lambda b, m: (b, m, 0)),
        ],
        out_specs=[
            pl.BlockSpec((1, 1, N), lambda b, m: (b, 0, 0)),
            pl.BlockSpec((1, 1, N), lambda b, m: (b, 0, 0)),
            pl.BlockSpec((1, tm, 1), lambda b, m: (b, m, 0)),
            pl.BlockSpec((1, tm, 1), lambda b, m: (b, m, 0)),
        ],
        out_shape=[
            jax.ShapeDtypeStruct((B, 1, N), f32),
            jax.ShapeDtypeStruct((B, 1, N), f32),
            jax.ShapeDtypeStruct((B, M, 1), f32),
            jax.ShapeDtypeStruct((B, M, 1), f32),
        ],
        scratch_shapes=[
            pltpu.VMEM((1, N), f32),
            pltpu.VMEM((3, N), f32),
        ],
    )(a2, a1t, n1, n1t, n2t, n2)
    return jnp.mean(d1) + jnp.mean(d2) + jnp.mean(nd1) + jnp.mean(nd2)


def kernel(xyz1, xyz2):
    return _chamfer(xyz1, xyz2)


# speedup vs baseline: 1.9592x; 1.0030x over previous
import functools

import jax
import jax.numpy as jnp
from jax.experimental import pallas as pl
from jax.experimental.pallas import tpu as pltpu

_EPS = 1e-12


def _chamfer_body(a2r_ref, a1t_ref, n1_ref, n1t_ref, n2t_ref, n2r_ref,
                  d1_ref, nd1_ref, d2_ref, nd2_ref,
                  accd_ref, accn_ref):
    mt = pl.program_id(1)
    nmt = pl.num_programs(1)

    a2r = a2r_ref[0]
    a1t = a1t_ref[0]
    n1 = n1_ref[0]
    n1t = n1t_ref[0]
    n2t = n2t_ref[0]
    n2r = n2r_ref[0]

    dT = jnp.dot(a2r, a1t, preferred_element_type=jnp.float32)

    rmin = jnp.min(dT, axis=0, keepdims=True)
    oh1 = (dT == rmin).astype(jnp.bfloat16)
    cand_n = jnp.dot(n2t, oh1, preferred_element_type=jnp.float32)

    @pl.when(mt == 0)
    def _():
        accd_ref[...] = rmin
        accn_ref[...] = cand_n

    @pl.when(mt > 0)
    def _():
        prev = accd_ref[...]
        upd = rmin < prev
        accd_ref[...] = jnp.where(upd, rmin, prev)
        accn_ref[...] = jnp.where(upd, cand_n, accn_ref[...])

    cmin = jnp.min(dT, axis=1, keepdims=True)
    oh2 = (dT == cmin).astype(jnp.bfloat16)
    tn2 = jnp.dot(oh2, n1, preferred_element_type=jnp.float32)

    d2_ref[0] = cmin

    inv2 = 1.0 / jnp.maximum(
        jnp.sqrt(jnp.sum(n2r * n2r, axis=1, keepdims=True)), _EPS)
    invt2 = 1.0 / jnp.maximum(
        jnp.sqrt(jnp.sum(tn2 * tn2, axis=1, keepdims=True)), _EPS)
    diff2 = n2r * inv2 - tn2 * invt2
    nd2_ref[0] = jnp.sum(diff2 * diff2, axis=1, keepdims=True)

    @pl.when(mt == nmt - 1)
    def _():
        an = accn_ref[...]
        d1_ref[0] = accd_ref[...]
        inv1 = 1.0 / jnp.maximum(
            jnp.sqrt(jnp.sum(n1t * n1t, axis=0, keepdims=True)), _EPS)
        invt1 = 1.0 / jnp.maximum(
            jnp.sqrt(jnp.sum(an * an, axis=0, keepdims=True)), _EPS)
        diff = n1t * inv1 - an * invt1
        nd1_ref[0] = jnp.sum(diff * diff, axis=0, keepdims=True)


@functools.partial(jax.jit, static_argnames=("tm",))
def _chamfer(xyz1, xyz2, tm=512):
    B, N, _ = xyz1.shape
    M = xyz2.shape[1]
    f32 = jnp.float32

    p1 = xyz1[:, :, :3]
    n1 = xyz1[:, :, 3:]
    p2 = xyz2[:, :, :3]
    n2 = xyz2[:, :, 3:]
    sq1 = jnp.sum(p1 * p1, axis=2, keepdims=True)
    sq2 = jnp.sum(p2 * p2, axis=2, keepdims=True)
    a1 = jnp.concatenate([p1, sq1, jnp.ones((B, N, 1), f32),
                          jnp.zeros((B, N, 3), f32)], axis=2)
    a2 = jnp.concatenate([-2.0 * p2, jnp.ones((B, M, 1), f32), sq2,
                          jnp.zeros((B, M, 3), f32)], axis=2)
    a1t = jnp.transpose(a1, (0, 2, 1))
    n1t = jnp.transpose(n1, (0, 2, 1))
    n1b = n1.astype(jnp.bfloat16)
    n2t = jnp.transpose(n2, (0, 2, 1)).astype(jnp.bfloat16)

    grid = (B, M // tm)
    d1, nd1, d2, nd2 = pl.pallas_call(
        _chamfer_body,
        grid=grid,
        in_specs=[
            pl.BlockSpec((1, tm, 8), lambda b, m: (b, m, 0)),
            pl.BlockSpec((1, 8, N), lambda b, m: (b, 0, 0)),
            pl.BlockSpec((1, N, 3), lambda b, m: (b, 0, 0)),
            pl.BlockSpec((1, 3, N), lambda b, m: (b, 0, 0)),
            pl.BlockSpec((1, 3, tm), lambda b, m: (b, 0, m)),
            pl.BlockSpec((1, tm, 3), lambda b, m: (b, m, 0)),
        ],
        out_specs=[
            pl.BlockSpec((1, 1, N), lambda b, m: (b, 0, 0)),
            pl.BlockSpec((1, 1, N), lambda b, m: (b, 0, 0)),
            pl.BlockSpec((1, tm, 1), lambda b, m: (b, m, 0)),
            pl.BlockSpec((1, tm, 1), lambda b, m: (b, m, 0)),
        ],
        out_shape=[
            jax.ShapeDtypeStruct((B, 1, N), f32),
            jax.ShapeDtypeStruct((B, 1, N), f32),
            jax.ShapeDtypeStruct((B, M, 1), f32),
            jax.ShapeDtypeStruct((B, M, 1), f32),
        ],
        scratch_shapes=[
            pltpu.VMEM((1, N), f32),
            pltpu.VMEM((3, N), f32),
        ],
    )(a2, a1t, n1b, n1t, n2t, n2)
    return jnp.mean(d1) + jnp.mean(d2) + jnp.mean(nd1) + jnp.mean(nd2)


def kernel(xyz1, xyz2):
    return _chamfer(xyz1, xyz2)


# speedup vs baseline: 2.0364x; 1.0394x over previous
import functools

import jax
import jax.numpy as jnp
from jax.experimental import pallas as pl
from jax.experimental.pallas import tpu as pltpu

_EPS = 1e-12


def _chamfer_body(a2r_ref, a1t_ref, n1_ref, n1t_ref, n2t_ref, n2r_ref,
                  d1_ref, nd1_ref, d2_ref, nd2_ref,
                  accd_ref, accn_ref):
    mt = pl.program_id(1)
    nmt = pl.num_programs(1)

    a2r = a2r_ref[0]
    a1t = a1t_ref[0]
    n1 = n1_ref[0]
    n1t = n1t_ref[0]
    n2t = n2t_ref[0]
    n2r = n2r_ref[0]

    dT = jnp.dot(a2r, a1t, preferred_element_type=jnp.float32)

    rmin = jnp.min(dT, axis=0, keepdims=True)
    oh1 = (dT == rmin).astype(jnp.bfloat16)
    cand_n = jnp.dot(n2t, oh1, preferred_element_type=jnp.float32)

    @pl.when(mt == 0)
    def _():
        accd_ref[...] = rmin
        accn_ref[...] = cand_n

    @pl.when(mt > 0)
    def _():
        prev = accd_ref[...]
        upd = rmin < prev
        accd_ref[...] = jnp.where(upd, rmin, prev)
        accn_ref[...] = jnp.where(upd, cand_n, accn_ref[...])

    cmin = jnp.min(dT, axis=1, keepdims=True)
    oh2 = (dT == cmin).astype(jnp.bfloat16)
    tn2 = jnp.dot(oh2, n1, preferred_element_type=jnp.float32)

    d2_ref[0] = cmin

    inv2 = 1.0 / jnp.maximum(
        jnp.sqrt(jnp.sum(n2r * n2r, axis=1, keepdims=True)), _EPS)
    invt2 = 1.0 / jnp.maximum(
        jnp.sqrt(jnp.sum(tn2 * tn2, axis=1, keepdims=True)), _EPS)
    diff2 = n2r * inv2 - tn2 * invt2
    nd2_ref[0] = jnp.sum(diff2 * diff2, axis=1, keepdims=True)

    @pl.when(mt == nmt - 1)
    def _():
        an = accn_ref[...]
        d1_ref[0] = accd_ref[...]
        inv1 = 1.0 / jnp.maximum(
            jnp.sqrt(jnp.sum(n1t * n1t, axis=0, keepdims=True)), _EPS)
        invt1 = 1.0 / jnp.maximum(
            jnp.sqrt(jnp.sum(an * an, axis=0, keepdims=True)), _EPS)
        diff = n1t * inv1 - an * invt1
        nd1_ref[0] = jnp.sum(diff * diff, axis=0, keepdims=True)


@functools.partial(jax.jit, static_argnames=("tm",))
def _chamfer(xyz1, xyz2, tm=1024):
    B, N, _ = xyz1.shape
    M = xyz2.shape[1]
    f32 = jnp.float32

    p1 = xyz1[:, :, :3]
    n1 = xyz1[:, :, 3:]
    p2 = xyz2[:, :, :3]
    n2 = xyz2[:, :, 3:]
    sq1 = jnp.sum(p1 * p1, axis=2, keepdims=True)
    sq2 = jnp.sum(p2 * p2, axis=2, keepdims=True)
    a1 = jnp.concatenate([p1, sq1, jnp.ones((B, N, 1), f32),
                          jnp.zeros((B, N, 3), f32)], axis=2)
    a2 = jnp.concatenate([-2.0 * p2, jnp.ones((B, M, 1), f32), sq2,
                          jnp.zeros((B, M, 3), f32)], axis=2)
    a1t = jnp.transpose(a1, (0, 2, 1))
    n1t = jnp.transpose(n1, (0, 2, 1))
    n1b = n1.astype(jnp.bfloat16)
    n2t = jnp.transpose(n2, (0, 2, 1)).astype(jnp.bfloat16)

    grid = (B, M // tm)
    d1, nd1, d2, nd2 = pl.pallas_call(
        _chamfer_body,
        grid=grid,
        in_specs=[
            pl.BlockSpec((1, tm, 8), lambda b, m: (b, m, 0)),
            pl.BlockSpec((1, 8, N), lambda b, m: (b, 0, 0)),
            pl.BlockSpec((1, N, 3), lambda b, m: (b, 0, 0)),
            pl.BlockSpec((1, 3, N), lambda b, m: (b, 0, 0)),
            pl.BlockSpec((1, 3, tm), lambda b, m: (b, 0, m)),
            pl.BlockSpec((1, tm, 3), lambda b, m: (b, m, 0)),
        ],
        out_specs=[
            pl.BlockSpec((1, 1, N), lambda b, m: (b, 0, 0)),
            pl.BlockSpec((1, 1, N), lambda b, m: (b, 0, 0)),
            pl.BlockSpec((1, tm, 1), lambda b, m: (b, m, 0)),
            pl.BlockSpec((1, tm, 1), lambda b, m: (b, m, 0)),
        ],
        out_shape=[
            jax.ShapeDtypeStruct((B, 1, N), f32),
            jax.ShapeDtypeStruct((B, 1, N), f32),
            jax.ShapeDtypeStruct((B, M, 1), f32),
            jax.ShapeDtypeStruct((B, M, 1), f32),
        ],
        scratch_shapes=[
            pltpu.VMEM((1, N), f32),
            pltpu.VMEM((3, N), f32),
        ],
    )(a2, a1t, n1b, n1t, n2t, n2)
    return jnp.mean(d1) + jnp.mean(d2) + jnp.mean(nd1) + jnp.mean(nd2)


def kernel(xyz1, xyz2):
    return _chamfer(xyz1, xyz2)
